# baseline (speedup 1.0000x reference)
import functools

import jax
import jax.numpy as jnp
from jax import lax
from jax.experimental import pallas as pl
from jax.experimental.pallas import tpu as pltpu
from jax.experimental.pallas import tpu_sc as plsc

D = 128
L = 16
KS = D // L
CH = 64
NBUF = 3
NR = 1000
NUM_CORES = 2
NUM_SUBCORES = 16


def _score_kernel(batch, per_w, num_cores):
    n_chunks = per_w // CH
    mesh = plsc.VectorSubcoreMesh(
        core_axis_name="c", subcore_axis_name="s",
        num_cores=NUM_CORES, num_subcores=NUM_SUBCORES)

    @functools.partial(
        pl.kernel,
        mesh=mesh,
        compiler_params=pltpu.CompilerParams(needs_layout_passes=False),
        out_type=jax.ShapeDtypeStruct((batch,), jnp.float32),
        scratch_types=(
            [pltpu.VMEM((per_w,), jnp.int32)] * 3
            + [pltpu.VMEM((CH, D), jnp.float32)] * (3 * NBUF)
            + [pltpu.VMEM((D + 8,), jnp.float32)]
            + [pltpu.VMEM((per_w,), jnp.float32)]
            + [pltpu.VMEM_SHARED((NR, D), jnp.float32)]
            + [pltpu.SemaphoreType.DMA] * (3 * NBUF + 5)
        ),
    )
    def k(heads_hbm, rels_hbm, tails_hbm, ent_hbm, rel_hbm, w_hbm, b_hbm,
          out_hbm, hidx, tidx, ridx,
          h0, h1, h2, t0, t1, t2, r0, r1, r2,
          p_v, out_v, rw_shared,
          sH0, sH1, sH2, sT0, sT1, sT2, sR0, sR1, sR2,
          si_h, si_t, si_r, s_w, s_b):
        sid = lax.axis_index("s")
        wid = sid * num_cores + lax.axis_index("c")
        base = wid * per_w
        ci_h = pltpu.async_copy(heads_hbm.at[pl.ds(base, per_w)], hidx, si_h)
        ci_t = pltpu.async_copy(tails_hbm.at[pl.ds(base, per_w)], tidx, si_t)
        ci_r = pltpu.async_copy(rels_hbm.at[pl.ds(base, per_w)], ridx, si_r)
        cw = pltpu.async_copy(w_hbm, p_v.at[pl.ds(0, D)], s_w)
        cb_ = pltpu.async_copy(b_hbm, p_v.at[pl.ds(D, 1)], s_b)

        hbufs = (h0, h1, h2)
        tbufs = (t0, t1, t2)
        rbufs = (r0, r1, r2)
        hsems = (sH0, sH1, sH2)
        tsems = (sT0, sT1, sT2)
        rsems = (sR0, sR1, sR2)

        def issue_ht(c, s):
            cb = c * CH
            ch = pltpu.async_copy(
                ent_hbm.at[hidx.at[pl.ds(cb, CH)]], hbufs[s], hsems[s])
            ct = pltpu.async_copy(
                ent_hbm.at[tidx.at[pl.ds(cb, CH)]], tbufs[s], tsems[s])
            return ch, ct

        def issue_r(c, s):
            cb = c * CH
            return pltpu.async_copy(
                rw_shared.at[ridx.at[pl.ds(cb, CH)]], rbufs[s], rsems[s])

        ci_h.wait()
        ci_t.wait()
        pend = [None] * NBUF
        for c in range(2):
            ch, ct = issue_ht(c, c)
            pend[c] = [ch, ct]

        cw.wait()
        cb_.wait()
        wk = [p_v[pl.ds(kk * L, L)] for kk in range(KS)]
        bvec = plsc.load_gather(p_v, [jnp.full((L,), D, jnp.int32)])

        FB = 64
        fb = sid * FB

        def _scale_rows(n1):
            def row_body(r, _):
                for kk in range(KS):
                    sl = pl.ds(kk * L, L)
                    r2[r, sl] = r2[r, sl] * wk[kk]
                return 0
            lax.fori_loop(0, n1, row_body, 0, unroll=2)

        @pl.when(sid < 15)
        def _():
            pltpu.sync_copy(rel_hbm.at[pl.ds(fb, FB)], r2.at[pl.ds(0, FB)])
            _scale_rows(FB)
            pltpu.sync_copy(r2.at[pl.ds(0, FB)], rw_shared.at[pl.ds(fb, FB)])

        @pl.when(sid == 15)
        def _():
            pltpu.sync_copy(rel_hbm.at[pl.ds(15 * FB, NR - 15 * FB)],
                            r2.at[pl.ds(0, NR - 15 * FB)])
            _scale_rows(NR - 15 * FB)
            pltpu.sync_copy(r2.at[pl.ds(0, NR - 15 * FB)],
                            rw_shared.at[pl.ds(15 * FB, NR - 15 * FB)])

        plsc.subcore_barrier()
        ci_r.wait()
        for c in range(2):
            pend[c].append(issue_r(c, c))

        iota = lax.iota(jnp.int32, L)
        zeros = jnp.zeros((L,), jnp.float32)

        def compute(c, s):
            hr, tr, rr = hbufs[s], tbufs[s], rbufs[s]

            def group_body(g):
                def triple_body(i, res):
                    r = g * L + i
                    acc = hr[r, pl.ds(0, L)] * tr[r, pl.ds(0, L)] \
                        * rr[r, pl.ds(0, L)]
                    for kk in range(1, KS):
                        sl = pl.ds(kk * L, L)
                        acc = acc + hr[r, sl] * tr[r, sl] * rr[r, sl]
                    ssum = jnp.sum(acc)
                    return jnp.where(iota == i, ssum, res)

                res = plsc.parallel_loop(0, L, carry=zeros)(triple_body)
                ob = pl.multiple_of(c * CH + g * L, 16)
                out_v[pl.ds(ob, L)] = res + bvec

            plsc.parallel_loop(0, CH // L)(group_body)

        for c in range(n_chunks):
            s = c % NBUF
            for cp in pend[s]:
                cp.wait()
            if c + 2 < n_chunks:
                s2 = (c + 2) % NBUF
                ch, ct = issue_ht(c + 2, s2)
                pend[s2] = [ch, ct, issue_r(c + 2, s2)]
            compute(c, s)

        pltpu.sync_copy(out_v, out_hbm.at[pl.ds(base, per_w)])

    return k


def kernel(heads, relations, tails, entity_table, relation_table, W, b):
    batch = heads.shape[0]
    nw = NUM_CORES * NUM_SUBCORES
    per_w = batch // nw
    k = _score_kernel(batch, per_w, NUM_CORES)
    return k(heads.astype(jnp.int32), relations.astype(jnp.int32),
             tails.astype(jnp.int32), entity_table, relation_table,
             W.reshape(D).astype(jnp.float32), b.astype(jnp.float32))


# speedup vs baseline: 1.0151x; 1.0151x over previous
import functools

import jax
import jax.numpy as jnp
from jax import lax
from jax.experimental import pallas as pl
from jax.experimental.pallas import tpu as pltpu
from jax.experimental.pallas import tpu_sc as plsc

D = 128
L = 16
KS = D // L
CH = 128
NR = 1000
NUM_CORES = 2
NUM_SUBCORES = 16


def _score_kernel(batch, per_w, num_cores):
    n_chunks = per_w // CH
    mesh = plsc.VectorSubcoreMesh(
        core_axis_name="c", subcore_axis_name="s",
        num_cores=NUM_CORES, num_subcores=NUM_SUBCORES)

    @functools.partial(
        pl.kernel,
        mesh=mesh,
        compiler_params=pltpu.CompilerParams(needs_layout_passes=False),
        out_type=jax.ShapeDtypeStruct((batch,), jnp.float32),
        scratch_types=[
            pltpu.VMEM((per_w,), jnp.int32),
            pltpu.VMEM((per_w,), jnp.int32),
            pltpu.VMEM((per_w,), jnp.int32),
            pltpu.VMEM((CH, D), jnp.float32),
            pltpu.VMEM((CH, D), jnp.float32),
            pltpu.VMEM((CH, D), jnp.float32),
            pltpu.VMEM((CH, D), jnp.float32),
            pltpu.VMEM((CH, D), jnp.float32),
            pltpu.VMEM((CH, D), jnp.float32),
            pltpu.VMEM((D + 8,), jnp.float32),
            pltpu.VMEM((per_w,), jnp.float32),
            pltpu.VMEM_SHARED((NR, D), jnp.float32),
            pltpu.SemaphoreType.DMA,
            pltpu.SemaphoreType.DMA,
            pltpu.SemaphoreType.DMA,
            pltpu.SemaphoreType.DMA,
            pltpu.SemaphoreType.DMA,
            pltpu.SemaphoreType.DMA,
            pltpu.SemaphoreType.DMA,
            pltpu.SemaphoreType.DMA,
        ],
    )
    def k(heads_hbm, rels_hbm, tails_hbm, ent_hbm, rel_hbm, w_hbm, b_hbm,
          out_hbm, hidx, tidx, ridx, hrows0, hrows1, trows0, trows1,
          rrows0, rrows1, p_v, out_v, rw_shared,
          sh0, sh1, st0, st1, sr0, sr1, s_w, s_b):
        sid = lax.axis_index("s")
        wid = sid * num_cores + lax.axis_index("c")
        base = wid * per_w
        ci_h = pltpu.async_copy(heads_hbm.at[pl.ds(base, per_w)], hidx, sh1)
        ci_t = pltpu.async_copy(tails_hbm.at[pl.ds(base, per_w)], tidx, st1)
        ci_r = pltpu.async_copy(rels_hbm.at[pl.ds(base, per_w)], ridx, sr1)
        cw = pltpu.async_copy(w_hbm, p_v.at[pl.ds(0, D)], s_w)
        cb_ = pltpu.async_copy(b_hbm, p_v.at[pl.ds(D, 1)], s_b)

        hbufs = (hrows0, hrows1)
        tbufs = (trows0, trows1)
        rbufs = (rrows0, rrows1)
        sems = ((sh0, st0, sr0), (sh1, st1, sr1))

        def issue_ht(c, s):
            cb = c * CH
            ch = pltpu.async_copy(
                ent_hbm.at[hidx.at[pl.ds(cb, CH)]], hbufs[s], sems[s][0])
            ct = pltpu.async_copy(
                ent_hbm.at[tidx.at[pl.ds(cb, CH)]], tbufs[s], sems[s][1])
            return ch, ct

        def issue_r(c, s):
            cb = c * CH
            return pltpu.async_copy(
                rw_shared.at[ridx.at[pl.ds(cb, CH)]], rbufs[s], sems[s][2])

        def issue(c, s):
            ch, ct = issue_ht(c, s)
            return ch, ct, issue_r(c, s)

        ci_h.wait()
        ci_t.wait()
        c0h, c0t = issue_ht(0, 0)

        cw.wait()
        cb_.wait()
        wk = [p_v[pl.ds(kk * L, L)] for kk in range(KS)]
        bvec = plsc.load_gather(p_v, [jnp.full((L,), D, jnp.int32)])

        FB = 64
        fb = sid * FB

        def _scale_rows(n1):
            def row_body(r, _):
                for kk in range(KS):
                    sl = pl.ds(kk * L, L)
                    rrows1[r, sl] = rrows1[r, sl] * wk[kk]
                return 0
            lax.fori_loop(0, n1, row_body, 0, unroll=2)

        @pl.when(sid < 15)
        def _():
            pltpu.sync_copy(rel_hbm.at[pl.ds(fb, FB)],
                            rrows1.at[pl.ds(0, FB)])
            _scale_rows(FB)
            pltpu.sync_copy(rrows1.at[pl.ds(0, FB)],
                            rw_shared.at[pl.ds(fb, FB)])

        @pl.when(sid == 15)
        def _():
            pltpu.sync_copy(rel_hbm.at[pl.ds(15 * FB, NR - 15 * FB)],
                            rrows1.at[pl.ds(0, NR - 15 * FB)])
            _scale_rows(NR - 15 * FB)
            pltpu.sync_copy(rrows1.at[pl.ds(0, NR - 15 * FB)],
                            rw_shared.at[pl.ds(15 * FB, NR - 15 * FB)])

        plsc.subcore_barrier()
        ci_r.wait()
        c0r = issue_r(0, 0)

        iota = lax.iota(jnp.int32, L)
        zeros = jnp.zeros((L,), jnp.float32)
        last = jnp.full((L,), L - 1, jnp.int32)

        def compute(c, s):
            hr, tr, rr = hbufs[s], tbufs[s], rbufs[s]

            def group_body(g):
                def triple_body(i, res):
                    r = g * L + i
                    acc = hr[r, pl.ds(0, L)] * tr[r, pl.ds(0, L)] \
                        * rr[r, pl.ds(0, L)]
                    for kk in range(1, KS):
                        sl = pl.ds(kk * L, L)
                        acc = acc + hr[r, sl] * tr[r, sl] * rr[r, sl]
                    tvec = lax.gather(
                        jnp.cumsum(acc), last[:, None],
                        dimension_numbers=lax.GatherDimensionNumbers(
                            offset_dims=(), collapsed_slice_dims=(0,),
                            start_index_map=(0,)),
                        slice_sizes=(1,),
                        mode=lax.GatherScatterMode.PROMISE_IN_BOUNDS)
                    return jnp.where(iota == i, tvec, res)

                res = plsc.parallel_loop(0, L, carry=zeros)(triple_body)
                ob = pl.multiple_of(c * CH + g * L, 16)
                out_v[pl.ds(ob, L)] = res + bvec

            plsc.parallel_loop(0, CH // L)(group_body)

        cps = (c0h, c0t, c0r)
        for c in range(n_chunks):
            s = c % 2
            for cp in cps:
                cp.wait()
            if c + 1 < n_chunks:
                nxt = issue(c + 1, (c + 1) % 2)
            compute(c, s)
            if c + 1 < n_chunks:
                cps = nxt

        pltpu.sync_copy(out_v, out_hbm.at[pl.ds(base, per_w)])

    return k


def kernel(heads, relations, tails, entity_table, relation_table, W, b):
    batch = heads.shape[0]
    nw = NUM_CORES * NUM_SUBCORES
    per_w = batch // nw
    k = _score_kernel(batch, per_w, NUM_CORES)
    return k(heads.astype(jnp.int32), relations.astype(jnp.int32),
             tails.astype(jnp.int32), entity_table, relation_table,
             W.reshape(D).astype(jnp.float32), b.astype(jnp.float32))
